# baseline (speedup 1.0000x reference)
import jax
import jax.numpy as jnp
from jax.experimental import pallas as pl
from jax.experimental.pallas import tpu as pltpu

B = 4
S = 4096
HIDDEN = 2048
N = B * S
CAPACITY = 0.7
TEMPERATURE = 0.5
LB_WEIGHT = 0.005
Z_LOSS_WEIGHT = 5e-06
K = max(1, min(int(CAPACITY * N), N))

BLK = 1024
NSTREAM = 2
NSTEP = N // (BLK * NSTREAM)
CHUNK = N // NSTREAM

R2 = 32
C2 = N // R2

_INT_MIN = -(2**31)


def _stage1(*refs):
    h_refs = refs[:NSTREAM]
    u_ref, w_ref, b_ref = refs[NSTREAM:NSTREAM + 3]
    noisy_ref, aux_ref = refs[NSTREAM + 3:NSTREAM + 5]
    acc_ref = refs[NSTREAM + 5]
    i = pl.program_id(0)
    w = w_ref[...]

    ps = None
    zs = None
    for s in range(NSTREAM):
        h = h_refs[s][...]
        scores = jax.lax.dot_general(
            h, w, (((1,), (0,)), ((), ())),
            preferred_element_type=jnp.float32)[:, 0:1]
        scores = scores + b_ref[0]
        u = u_ref[0, s]
        gumbel = -jnp.log(-jnp.log(u + 1e-10) + 1e-10)
        noisy_ref[s] = (scores + gumbel) / TEMPERATURE
        p = jax.nn.sigmoid(scores)
        z = scores * scores
        ps = p if ps is None else ps + p
        zs = z if zs is None else zs + z

    prev_p = jnp.where(i == 0, jnp.zeros_like(ps), acc_ref[...][:, 0:1])
    prev_z = jnp.where(i == 0, jnp.zeros_like(zs), acc_ref[...][:, 1:2])
    acc_ref[:, 0:1] = prev_p + ps
    acc_ref[:, 1:2] = prev_z + zs

    @pl.when(i == NSTEP - 1)
    def _():
        p = jnp.sum(acc_ref[...][:, 0:1]) / N
        z = jnp.sum(acc_ref[...][:, 1:2]) / N
        f = jnp.float32(K) / jnp.float32(N)
        lb = (f - CAPACITY) ** 2 + (p - CAPACITY) ** 2
        aux_ref[0] = LB_WEIGHT * lb + Z_LOSS_WEIGHT * z


def _stage2(noisy_ref, mask_ref):
    x = noisy_ref[...]
    b = jax.lax.bitcast_convert_type(x, jnp.int32)
    ku = b ^ ((b >> 31) | _INT_MIN)

    def radix_body(t, carry):
        prefix, remk, cand = carry
        bit = 31 - t
        bits1 = (ku >> bit) & 1
        ones = cand & bits1
        c1 = jnp.sum(ones)
        take = c1 >= remk
        prefix = jnp.where(take, prefix | (jnp.int32(1) << bit), prefix)
        cand = jnp.where(take, ones, cand - ones)
        remk = jnp.where(take, remk, remk - c1)
        return prefix, remk, cand

    tkey, need_eq, _ = jax.lax.fori_loop(
        0, 32, radix_body,
        (jnp.int32(0), jnp.int32(K), jnp.ones(x.shape, dtype=jnp.int32)))

    ks = ku ^ _INT_MIN
    ts = tkey ^ _INT_MIN
    gt = ks > ts
    eq = ku == tkey

    idx = (jax.lax.broadcasted_iota(jnp.int32, x.shape, 0) * C2
           + jax.lax.broadcasted_iota(jnp.int32, x.shape, 1))

    def bs_body(t, lohi):
        lo, hi = lohi
        mid = (lo + hi) // 2
        cnt = jnp.sum((eq & (idx < mid)).astype(jnp.int32))
        ge = cnt >= need_eq
        return jnp.where(ge, lo, mid), jnp.where(ge, mid, hi)

    _, cut = jax.lax.fori_loop(0, 14, bs_body, (jnp.int32(0), jnp.int32(N)))
    sel = gt | (eq & (idx < cut))
    mask_ref[...] = sel.astype(jnp.int8)


def kernel(hidden_states, active_mask, router_w, router_b, gumbel_u):
    del active_mask
    h = hidden_states.reshape(N, HIDDEN)
    u = gumbel_u.reshape(NSTREAM, NSTEP, BLK, 1).transpose(1, 0, 2, 3)
    w128 = jnp.pad(router_w.T, ((0, 0), (0, 127)))

    def mk_map(s):
        return lambda i: (i + s * NSTEP, 0)

    noisy, aux = pl.pallas_call(
        _stage1,
        grid=(NSTEP,),
        in_specs=(
            [pl.BlockSpec((BLK, HIDDEN), mk_map(s)) for s in range(NSTREAM)]
            + [
                pl.BlockSpec((1, NSTREAM, BLK, 1), lambda i: (i, 0, 0, 0)),
                pl.BlockSpec((HIDDEN, 128), lambda i: (0, 0)),
                pl.BlockSpec(memory_space=pltpu.SMEM),
            ]
        ),
        out_specs=[
            pl.BlockSpec((NSTREAM, BLK, 1), lambda i: (0, i, 0)),
            pl.BlockSpec(memory_space=pltpu.SMEM),
        ],
        out_shape=[
            jax.ShapeDtypeStruct((NSTREAM, CHUNK, 1), jnp.float32),
            jax.ShapeDtypeStruct((1,), jnp.float32),
        ],
        scratch_shapes=[pltpu.VMEM((BLK, 2), jnp.float32)],
    )(*([h] * NSTREAM), u, w128, router_b)

    mask8 = pl.pallas_call(
        _stage2,
        out_shape=jax.ShapeDtypeStruct((R2, C2), jnp.int8),
    )(noisy.reshape(R2, C2))

    ffn_mask = mask8.reshape(B, S).astype(bool)
    return ffn_mask, aux[0]


# speedup vs baseline: 1.1229x; 1.1229x over previous
import jax
import jax.numpy as jnp
from jax.experimental import pallas as pl
from jax.experimental.pallas import tpu as pltpu

B = 4
S = 4096
HIDDEN = 2048
N = B * S
CAPACITY = 0.7
TEMPERATURE = 0.5
LB_WEIGHT = 0.005
Z_LOSS_WEIGHT = 5e-06
K = max(1, min(int(CAPACITY * N), N))

BLK = 1024
NSTREAM = 2
NSTEP = N // (BLK * NSTREAM)
CHUNK = N // NSTREAM

R2 = 32
C2 = N // R2

_INT_MIN = -(2**31)


def _stage1(*refs):
    h_refs = refs[:NSTREAM]
    w_ref, b_ref = refs[NSTREAM:NSTREAM + 2]
    scores_ref = refs[NSTREAM + 2]
    w = w_ref[...]
    for s in range(NSTREAM):
        h = h_refs[s][...]
        scores = jax.lax.dot_general(
            h, w, (((1,), (0,)), ((), ())),
            preferred_element_type=jnp.float32)[:, 0:1]
        scores_ref[s] = scores + b_ref[0]


def _stage2(scores_ref, u_ref, mask_ref, aux_ref):
    scores = scores_ref[...]
    u = u_ref[...]
    gumbel = -jnp.log(-jnp.log(u + 1e-10) + 1e-10)
    x = (scores + gumbel) / TEMPERATURE

    p = jnp.sum(jax.nn.sigmoid(scores)) / N
    z = jnp.sum(scores * scores) / N
    f = jnp.float32(K) / jnp.float32(N)
    lb = (f - CAPACITY) ** 2 + (p - CAPACITY) ** 2
    aux_ref[0] = LB_WEIGHT * lb + Z_LOSS_WEIGHT * z

    b = jax.lax.bitcast_convert_type(x, jnp.int32)
    ku = b ^ ((b >> 31) | _INT_MIN)

    one11 = jnp.ones((1, 1), jnp.int32)

    def radix_body(t, carry):
        prefix, remk, cand = carry
        bit = 31 - t
        bits1 = (ku >> bit) & 1
        ones = cand & bits1
        c1 = jnp.sum(ones).reshape(1, 1)
        take = c1 >= remk
        prefix = jnp.where(take, prefix | (one11 << bit), prefix)
        cand = jnp.where(take, ones, cand - ones)
        remk = jnp.where(take, remk, remk - c1)
        return prefix, remk, cand

    tkey, need_eq, _ = jax.lax.fori_loop(
        0, 32, radix_body,
        (jnp.zeros((1, 1), jnp.int32), jnp.full((1, 1), K, jnp.int32),
         jnp.ones(x.shape, dtype=jnp.int32)))

    ks = ku ^ _INT_MIN
    ts = tkey ^ _INT_MIN
    gt = ks > ts
    eq = ku == tkey

    idx = (jax.lax.broadcasted_iota(jnp.int32, x.shape, 0) * C2
           + jax.lax.broadcasted_iota(jnp.int32, x.shape, 1))

    def bs_body(t, lohi):
        lo, hi = lohi
        mid = (lo + hi) >> 1
        cnt = jnp.sum(jnp.where(eq & (idx < mid), 1, 0)).reshape(1, 1)
        ge = cnt >= need_eq
        return jnp.where(ge, lo, mid), jnp.where(ge, mid, hi)

    _, cut = jax.lax.fori_loop(
        0, 14, bs_body,
        (jnp.zeros((1, 1), jnp.int32), jnp.full((1, 1), N, jnp.int32)))
    sel = gt | (eq & (idx < cut))
    mask_ref[...] = sel.astype(jnp.int8)


def kernel(hidden_states, active_mask, router_w, router_b, gumbel_u):
    del active_mask
    h = hidden_states.reshape(N, HIDDEN)
    w128 = jnp.pad(router_w.T, ((0, 0), (0, 127)))

    def mk_map(s):
        return lambda i: (i + s * NSTEP, 0)

    scores = pl.pallas_call(
        _stage1,
        grid=(NSTEP,),
        in_specs=(
            [pl.BlockSpec((BLK, HIDDEN), mk_map(s)) for s in range(NSTREAM)]
            + [
                pl.BlockSpec((HIDDEN, 128), lambda i: (0, 0)),
                pl.BlockSpec(memory_space=pltpu.SMEM),
            ]
        ),
        out_specs=pl.BlockSpec((NSTREAM, BLK, 1), lambda i: (0, i, 0)),
        out_shape=jax.ShapeDtypeStruct((NSTREAM, CHUNK, 1), jnp.float32),
    )(*([h] * NSTREAM), w128, router_b)

    mask8, aux = pl.pallas_call(
        _stage2,
        out_shape=[
            jax.ShapeDtypeStruct((R2, C2), jnp.int8),
            jax.ShapeDtypeStruct((1,), jnp.float32),
        ],
        out_specs=[
            pl.BlockSpec((R2, C2), lambda: (0, 0)),
            pl.BlockSpec(memory_space=pltpu.SMEM),
        ],
    )(scores.reshape(R2, C2), gumbel_u.reshape(R2, C2))

    ffn_mask = mask8.reshape(B, S).astype(bool)
    return ffn_mask, aux[0]


# speedup vs baseline: 1.2956x; 1.1538x over previous
import jax
import jax.numpy as jnp
from jax.experimental import pallas as pl
from jax.experimental.pallas import tpu as pltpu

B = 4
S = 4096
HIDDEN = 2048
N = B * S
CAPACITY = 0.7
TEMPERATURE = 0.5
LB_WEIGHT = 0.005
Z_LOSS_WEIGHT = 5e-06
K = max(1, min(int(CAPACITY * N), N))

BLK = 1024
NSTREAM = 2
NSTEP = N // (BLK * NSTREAM)
SUBROWS = BLK // 128

RT = N // 128

_INT_MIN = -(2**31)


def _fused(*refs):
    h_refs = refs[:NSTREAM]
    u_ref, w_ref, b_ref = refs[NSTREAM:NSTREAM + 3]
    mask_ref, aux_ref = refs[NSTREAM + 3:NSTREAM + 5]
    sacc_ref = refs[NSTREAM + 5]
    i = pl.program_id(0)
    w = w_ref[...]
    for s in range(NSTREAM):
        h = h_refs[s][...]
        scores = jax.lax.dot_general(
            h, w, (((1,), (0,)), ((), ())),
            preferred_element_type=jnp.float32)[:, 0:1]
        scores = scores + b_ref[0]
        sacc_ref[pl.ds(i * SUBROWS + s * (NSTEP * SUBROWS), SUBROWS), :] = (
            scores.reshape(SUBROWS, 128))

    @pl.when(i == NSTEP - 1)
    def _():
        scores = sacc_ref[...]
        u = u_ref[...]
        gumbel = -jnp.log(-jnp.log(u + 1e-10) + 1e-10)
        x = (scores + gumbel) / TEMPERATURE

        p = jnp.sum(jax.nn.sigmoid(scores)) / N
        z = jnp.sum(scores * scores) / N
        f = jnp.float32(K) / jnp.float32(N)
        lb = (f - CAPACITY) ** 2 + (p - CAPACITY) ** 2
        aux_ref[0] = LB_WEIGHT * lb + Z_LOSS_WEIGHT * z

        bb = jax.lax.bitcast_convert_type(x, jnp.int32)
        ku = bb ^ ((bb >> 31) | _INT_MIN)

        one11 = jnp.ones((1, 1), jnp.int32)

        def radix_body(t, carry):
            prefix, remk, cand = carry
            bit = 31 - t
            bits1 = (ku >> bit) & 1
            ones = cand & bits1
            c1 = jnp.sum(ones).reshape(1, 1)
            take = c1 >= remk
            prefix = jnp.where(take, prefix | (one11 << bit), prefix)
            cand = jnp.where(take, ones, cand - ones)
            remk = jnp.where(take, remk, remk - c1)
            return prefix, remk, cand

        tkey, need_eq, _ = jax.lax.fori_loop(
            0, 32, radix_body,
            (jnp.zeros((1, 1), jnp.int32), jnp.full((1, 1), K, jnp.int32),
             jnp.ones(x.shape, dtype=jnp.int32)))

        ks = ku ^ _INT_MIN
        ts = tkey ^ _INT_MIN
        gt = ks > ts
        eq = ku == tkey

        idx = (jax.lax.broadcasted_iota(jnp.int32, x.shape, 0) * 128
               + jax.lax.broadcasted_iota(jnp.int32, x.shape, 1))

        def bs_body(t, lohi):
            lo, hi = lohi
            mid = (lo + hi) >> 1
            cnt = jnp.sum(jnp.where(eq & (idx < mid), 1, 0)).reshape(1, 1)
            ge = cnt >= need_eq
            return jnp.where(ge, lo, mid), jnp.where(ge, mid, hi)

        _, cut = jax.lax.fori_loop(
            0, 14, bs_body,
            (jnp.zeros((1, 1), jnp.int32), jnp.full((1, 1), N, jnp.int32)))
        sel = gt | (eq & (idx < cut))
        mask_ref[...] = sel.astype(jnp.int8)


def kernel(hidden_states, active_mask, router_w, router_b, gumbel_u):
    del active_mask
    h = hidden_states.reshape(N, HIDDEN)
    w128 = jnp.pad(router_w.T, ((0, 0), (0, 127)))

    def mk_map(s):
        return lambda i: (i + s * NSTEP, 0)

    mask8, aux = pl.pallas_call(
        _fused,
        grid=(NSTEP,),
        in_specs=(
            [pl.BlockSpec((BLK, HIDDEN), mk_map(s)) for s in range(NSTREAM)]
            + [
                pl.BlockSpec((RT, 128), lambda i: (0, 0)),
                pl.BlockSpec((HIDDEN, 128), lambda i: (0, 0)),
                pl.BlockSpec(memory_space=pltpu.SMEM),
            ]
        ),
        out_specs=[
            pl.BlockSpec((RT, 128), lambda i: (0, 0)),
            pl.BlockSpec(memory_space=pltpu.SMEM),
        ],
        out_shape=[
            jax.ShapeDtypeStruct((RT, 128), jnp.int8),
            jax.ShapeDtypeStruct((1,), jnp.float32),
        ],
        scratch_shapes=[pltpu.VMEM((RT, 128), jnp.float32)],
    )(*([h] * NSTREAM), gumbel_u.reshape(RT, 128), w128, router_b)

    ffn_mask = mask8.reshape(B, S).astype(bool)
    return ffn_mask, aux[0]
